# speedup vs baseline: 1.4220x; 1.4220x over previous
import functools

import jax
import jax.numpy as jnp
from jax.experimental import pallas as pl
from jax.experimental.pallas import tpu as pltpu

_D_MODEL = 768
_N_EXPERTS = 8
_D_FF = 768


def _router_body(x_ref, gw_ref, dr_ref):
    x = x_ref[...]
    gw = gw_ref[...]
    logits = jnp.dot(x, gw, preferred_element_type=jnp.float32)
    e_iota = jax.lax.broadcasted_iota(jnp.int32, logits.shape, 1)
    idx1 = jnp.argmax(logits, axis=1)
    one1 = e_iota == idx1[:, None]
    m1 = jnp.max(logits, axis=1, keepdims=True)
    neg = jnp.finfo(jnp.float32).min
    l2 = jnp.where(one1, neg, logits)
    idx2 = jnp.argmax(l2, axis=1)
    one2 = e_iota == idx2[:, None]
    m2 = jnp.max(l2, axis=1, keepdims=True)
    w1 = 1.0 / (1.0 + jnp.exp(m2 - m1))
    w2 = 1.0 - w1
    dr_ref[...] = jnp.where(one1, w1, jnp.where(one2, w2, 0.0))


def _moe_body(x_ref, dr_ref, gu_ref, dn_ref, out_ref):
    e = pl.program_id(1)
    x = x_ref[...]
    h = jnp.dot(x, gu_ref[0], preferred_element_type=jnp.float32)
    gate = h[:, :_D_FF]
    up = h[:, _D_FF:]
    act = (gate * jax.lax.logistic(gate) * up).astype(jnp.bfloat16)
    y = jnp.dot(act, dn_ref[0], preferred_element_type=jnp.float32)
    dr = dr_ref[...]
    e_iota = jax.lax.broadcasted_iota(jnp.int32, dr.shape, 1)
    w = jnp.sum(jnp.where(e_iota == e, dr, 0.0), axis=1, keepdims=True)
    contrib = y * w

    @pl.when(e == 0)
    def _():
        out_ref[...] = contrib

    @pl.when(e > 0)
    def _():
        out_ref[...] += contrib


@jax.jit
def kernel(hidden_states, gate_w, gate_up_proj, down_proj):
    batch, seq, d = hidden_states.shape
    T = batch * seq
    x = hidden_states.reshape(T, d)

    dense_router = pl.pallas_call(
        _router_body,
        out_shape=jax.ShapeDtypeStruct((T, _N_EXPERTS), jnp.float32),
    )(x, gate_w)

    xb = x.astype(jnp.bfloat16)
    gub = gate_up_proj.astype(jnp.bfloat16)
    dnb = down_proj.astype(jnp.bfloat16)

    TB = 2048
    n_tb = T // TB
    out = pl.pallas_call(
        _moe_body,
        grid=(n_tb, _N_EXPERTS),
        in_specs=[
            pl.BlockSpec((TB, d), lambda tb, e: (tb, 0)),
            pl.BlockSpec((TB, _N_EXPERTS), lambda tb, e: (tb, 0)),
            pl.BlockSpec((1, d, 2 * _D_FF), lambda tb, e: (e, 0, 0)),
            pl.BlockSpec((1, _D_FF, d), lambda tb, e: (e, 0, 0)),
        ],
        out_specs=pl.BlockSpec((TB, d), lambda tb, e: (tb, 0)),
        out_shape=jax.ShapeDtypeStruct((T, d), jnp.float32),
    )(xb, dense_router, gub, dnb)

    return out.reshape(batch, seq, d)


# speedup vs baseline: 1.9392x; 1.3638x over previous
import functools

import jax
import jax.numpy as jnp
from jax.experimental import pallas as pl
from jax.experimental.pallas import tpu as pltpu

_D_MODEL = 768
_N_EXPERTS = 8
_D_FF = 768


def _router_body(x_ref, gw_ref, dr_ref):
    x = x_ref[...]
    gw = gw_ref[...]
    logits = jnp.dot(x, gw, preferred_element_type=jnp.float32)
    e_iota = jax.lax.broadcasted_iota(jnp.int32, logits.shape, 1)
    idx1 = jnp.argmax(logits, axis=1)
    one1 = e_iota == idx1[:, None]
    m1 = jnp.max(logits, axis=1, keepdims=True)
    neg = jnp.finfo(jnp.float32).min
    l2 = jnp.where(one1, neg, logits)
    idx2 = jnp.argmax(l2, axis=1)
    one2 = e_iota == idx2[:, None]
    m2 = jnp.max(l2, axis=1, keepdims=True)
    w1 = 1.0 / (1.0 + jnp.exp(m2 - m1))
    w2 = 1.0 - w1
    dr_ref[...] = jnp.where(one1, w1, jnp.where(one2, w2, 0.0))


def _moe_body(x_ref, dr_ref, gu_ref, dn_ref, out_ref):
    e = pl.program_id(1)
    x = x_ref[...].astype(jnp.bfloat16)
    gu = gu_ref[0].astype(jnp.bfloat16)
    h = jnp.dot(x, gu, preferred_element_type=jnp.float32)
    gate = h[:, :_D_FF]
    up = h[:, _D_FF:]
    act = (gate * jax.lax.logistic(gate) * up).astype(jnp.bfloat16)
    dn = dn_ref[0].astype(jnp.bfloat16)
    y = jnp.dot(act, dn, preferred_element_type=jnp.float32)
    dr = dr_ref[...]
    e_iota = jax.lax.broadcasted_iota(jnp.int32, dr.shape, 1)
    w = jnp.sum(jnp.where(e_iota == e, dr, 0.0), axis=1, keepdims=True)
    contrib = y * w

    @pl.when(e == 0)
    def _():
        out_ref[...] = contrib

    @pl.when(e > 0)
    def _():
        out_ref[...] += contrib


@jax.jit
def kernel(hidden_states, gate_w, gate_up_proj, down_proj):
    batch, seq, d = hidden_states.shape
    T = batch * seq
    x = hidden_states.reshape(T, d)

    dense_router = pl.pallas_call(
        _router_body,
        out_shape=jax.ShapeDtypeStruct((T, _N_EXPERTS), jnp.float32),
    )(x, gate_w)

    TB = 2048
    n_tb = T // TB
    out = pl.pallas_call(
        _moe_body,
        grid=(n_tb, _N_EXPERTS),
        in_specs=[
            pl.BlockSpec((TB, d), lambda tb, e: (tb, 0)),
            pl.BlockSpec((TB, _N_EXPERTS), lambda tb, e: (tb, 0)),
            pl.BlockSpec((1, d, 2 * _D_FF), lambda tb, e: (e, 0, 0)),
            pl.BlockSpec((1, _D_FF, d), lambda tb, e: (e, 0, 0)),
        ],
        out_specs=pl.BlockSpec((TB, d), lambda tb, e: (tb, 0)),
        out_shape=jax.ShapeDtypeStruct((T, d), jnp.float32),
    )(x, dense_router, gate_up_proj, down_proj)

    return out.reshape(batch, seq, d)


# speedup vs baseline: 2.0411x; 1.0526x over previous
import functools

import jax
import jax.numpy as jnp
from jax.experimental import pallas as pl
from jax.experimental.pallas import tpu as pltpu

_D_MODEL = 768
_N_EXPERTS = 8
_D_FF = 768


def _router_weights(x, gw):
    logits = jnp.dot(x, gw, preferred_element_type=jnp.float32)
    e_iota = jax.lax.broadcasted_iota(jnp.int32, logits.shape, 1)
    idx1 = jnp.argmax(logits, axis=1)
    one1 = e_iota == idx1[:, None]
    m1 = jnp.max(logits, axis=1, keepdims=True)
    neg = jnp.finfo(jnp.float32).min
    l2 = jnp.where(one1, neg, logits)
    idx2 = jnp.argmax(l2, axis=1)
    one2 = e_iota == idx2[:, None]
    m2 = jnp.max(l2, axis=1, keepdims=True)
    w1 = 1.0 / (1.0 + jnp.exp(m2 - m1))
    w2 = 1.0 - w1
    return jnp.where(one1, w1, jnp.where(one2, w2, 0.0))


def _moe_body(x_ref, gw_ref, gu_ref, dn_ref, out_ref, dr_ref):
    e = pl.program_id(0)

    @pl.when(e == 0)
    def _():
        dr_ref[...] = _router_weights(x_ref[...], gw_ref[...])

    x = x_ref[...].astype(jnp.bfloat16)
    gu = gu_ref[0].astype(jnp.bfloat16)
    h = jnp.dot(x, gu, preferred_element_type=jnp.float32)
    gate = h[:, :_D_FF]
    up = h[:, _D_FF:]
    act = (gate * jax.lax.logistic(gate) * up).astype(jnp.bfloat16)
    dn = dn_ref[0].astype(jnp.bfloat16)
    y = jnp.dot(act, dn, preferred_element_type=jnp.float32)
    dr = dr_ref[...]
    e_iota = jax.lax.broadcasted_iota(jnp.int32, dr.shape, 1)
    w = jnp.sum(jnp.where(e_iota == e, dr, 0.0), axis=1, keepdims=True)
    contrib = y * w

    @pl.when(e == 0)
    def _():
        out_ref[...] = contrib

    @pl.when(e > 0)
    def _():
        out_ref[...] += contrib


@jax.jit
def kernel(hidden_states, gate_w, gate_up_proj, down_proj):
    batch, seq, d = hidden_states.shape
    T = batch * seq
    x = hidden_states.reshape(T, d)

    out = pl.pallas_call(
        _moe_body,
        grid=(_N_EXPERTS,),
        in_specs=[
            pl.BlockSpec((T, d), lambda e: (0, 0)),
            pl.BlockSpec((d, _N_EXPERTS), lambda e: (0, 0)),
            pl.BlockSpec((1, d, 2 * _D_FF), lambda e: (e, 0, 0)),
            pl.BlockSpec((1, _D_FF, d), lambda e: (e, 0, 0)),
        ],
        out_specs=pl.BlockSpec((T, d), lambda e: (0, 0)),
        out_shape=jax.ShapeDtypeStruct((T, d), jnp.float32),
        scratch_shapes=[pltpu.VMEM((T, _N_EXPERTS), jnp.float32)],
    )(x, gate_w, gate_up_proj, down_proj)

    return out.reshape(batch, seq, d)
